# Initial kernel scaffold; baseline (speedup 1.0000x reference)
import functools

import jax
import jax.numpy as jnp
from jax import lax
from jax.experimental import pallas as pl
from jax.experimental.pallas import tpu as pltpu
from jax.experimental.pallas import tpu_sc as plsc

N_NODES = 10000
N_EDGES = 320000
DIM = 128
N_GRAPHS = 64
N_OUT = 16

NUM_CORES = 2
NUM_SUBCORES = 16
NUM_TILES = NUM_CORES * NUM_SUBCORES
EDGES_PER_TILE = N_EDGES // NUM_TILES
CHUNK = 80
N_CHUNKS = EDGES_PER_TILE // CHUNK
ROWS_PER_SUBCORE = N_NODES // NUM_SUBCORES


def _sc_agg_body(h_hbm, src_hbm, dst_hbm, zeros_hbm, out_hbm,
                 src_v, dst_v, rows_v, agg_sh, sem):
    c = lax.axis_index("c")
    s = lax.axis_index("s")
    wid = c * NUM_SUBCORES + s
    r0 = s * ROWS_PER_SUBCORE

    pltpu.sync_copy(zeros_hbm.at[pl.ds(r0, ROWS_PER_SUBCORE)],
                    agg_sh.at[pl.ds(r0, ROWS_PER_SUBCORE)])
    plsc.subcore_barrier()

    e_base = wid * EDGES_PER_TILE

    def body(j, carry):
        e0 = e_base + j * CHUNK
        pltpu.sync_copy(src_hbm.at[pl.ds(e0, CHUNK)], src_v)
        pltpu.sync_copy(dst_hbm.at[pl.ds(e0, CHUNK)], dst_v)
        pltpu.async_copy(h_hbm.at[src_v], rows_v, sem).wait()
        pltpu.sync_copy(rows_v, agg_sh.at[dst_v], add=True)
        return carry

    lax.fori_loop(0, N_CHUNKS, body, 0)
    plsc.subcore_barrier()

    pltpu.sync_copy(agg_sh.at[pl.ds(r0, ROWS_PER_SUBCORE)],
                    out_hbm.at[c, pl.ds(r0, ROWS_PER_SUBCORE)])


_sc_agg = pl.kernel(
    _sc_agg_body,
    out_type=jax.ShapeDtypeStruct((NUM_CORES, N_NODES, DIM), jnp.float32),
    mesh=plsc.VectorSubcoreMesh(core_axis_name="c", subcore_axis_name="s",
                                num_cores=NUM_CORES, num_subcores=NUM_SUBCORES),
    scratch_types=[
        pltpu.VMEM((CHUNK,), jnp.int32),
        pltpu.VMEM((CHUNK,), jnp.int32),
        pltpu.VMEM((CHUNK, DIM), jnp.float32),
        pltpu.VMEM_SHARED((N_NODES, DIM), jnp.float32),
        pltpu.SemaphoreType.DMA,
    ],
)


def _bn(z, g, b):
    m = jnp.mean(z, axis=0, keepdims=True)
    v = jnp.mean((z - m) * (z - m), axis=0, keepdims=True)
    return (z - m) * lax.rsqrt(v + 1e-5) * g + b


def _tc_layer_body(h_ref, agg_ref, w1_ref, b1_ref, g1_ref, be1_ref,
                   w2_ref, b2_ref, g2_ref, be2_ref, out_ref):
    a = agg_ref[...]
    z = h_ref[...] + a[0] + a[1]
    z = jnp.dot(z, w1_ref[...], preferred_element_type=jnp.float32) + b1_ref[...]
    z = jnp.maximum(_bn(z, g1_ref[...], be1_ref[...]), 0.0)
    z = jnp.dot(z, w2_ref[...], preferred_element_type=jnp.float32) + b2_ref[...]
    z = jnp.maximum(_bn(z, g2_ref[...], be2_ref[...]), 0.0)
    out_ref[...] = z


def _tc_final_body(h_ref, agg_ref, w1_ref, b1_ref, g1_ref, be1_ref,
                   w2_ref, b2_ref, g2_ref, be2_ref,
                   batch_ref, clsw_ref, clsb_ref, out_ref):
    a = agg_ref[...]
    z = h_ref[...] + a[0] + a[1]
    z = jnp.dot(z, w1_ref[...], preferred_element_type=jnp.float32) + b1_ref[...]
    z = jnp.maximum(_bn(z, g1_ref[...], be1_ref[...]), 0.0)
    z = jnp.dot(z, w2_ref[...], preferred_element_type=jnp.float32) + b2_ref[...]
    z = _bn(z, g2_ref[...], be2_ref[...])

    ids = lax.broadcasted_iota(jnp.int32, (N_NODES, N_GRAPHS), 1)
    onehot = (batch_ref[...] == ids).astype(jnp.float32)
    dnums = (((0,), (0,)), ((), ()))
    sums = lax.dot_general(onehot, z, dnums,
                           preferred_element_type=jnp.float32)
    cnts = lax.dot_general(onehot, jnp.ones((N_NODES, 1), jnp.float32),
                           dnums, preferred_element_type=jnp.float32)
    readout = sums / jnp.maximum(cnts, 1.0)
    out_ref[...] = (jnp.dot(readout, clsw_ref[...],
                            preferred_element_type=jnp.float32)
                    + clsb_ref[...])


_tc_layer = pl.pallas_call(
    _tc_layer_body,
    out_shape=jax.ShapeDtypeStruct((N_NODES, DIM), jnp.float32),
)

_tc_final = pl.pallas_call(
    _tc_final_body,
    out_shape=jax.ShapeDtypeStruct((N_GRAPHS, N_OUT), jnp.float32),
)


def kernel(x, edge_index, batch, params):
    src = edge_index[0]
    dst = edge_index[1]
    zeros = jnp.zeros((N_NODES, DIM), jnp.float32)
    batch2d = batch.reshape(N_NODES, 1).astype(jnp.int32)

    h = x
    layers = params["layers"]
    out = None
    for i, p in enumerate(layers):
        aggs = _sc_agg(h, src, dst, zeros)
        w = (p["W1"], p["b1"].reshape(1, -1), p["g1"].reshape(1, -1),
             p["be1"].reshape(1, -1), p["W2"], p["b2"].reshape(1, -1),
             p["g2"].reshape(1, -1), p["be2"].reshape(1, -1))
        if i != len(layers) - 1:
            h = _tc_layer(h, aggs, *w)
        else:
            out = _tc_final(h, aggs, *w, batch2d, params["cls_W"],
                            params["cls_b"].reshape(1, -1))
    return out


# speedup vs baseline: 4.5524x; 4.5524x over previous
import functools

import jax
import jax.numpy as jnp
from jax import lax
from jax.experimental import pallas as pl
from jax.experimental.pallas import tpu as pltpu
from jax.experimental.pallas import tpu_sc as plsc

N_NODES = 10000
N_EDGES = 320000
DIM = 128
N_GRAPHS = 64
N_OUT = 16

NUM_CORES = 2
NUM_SUBCORES = 16
NUM_TILES = NUM_CORES * NUM_SUBCORES
EDGES_PER_TILE = N_EDGES // NUM_TILES
CHUNK = 80
N_CHUNKS = EDGES_PER_TILE // CHUNK
ROWS_PER_SUBCORE = 624
TAIL_ROW0 = ROWS_PER_SUBCORE * NUM_SUBCORES
TAIL_ROWS = N_NODES - TAIL_ROW0


def _sc_agg_body(h_hbm, src_hbm, dst_hbm, zeros_hbm, out_hbm,
                 src_v, dst_v, rows_v, agg_sh, sem):
    c = lax.axis_index("c")
    s = lax.axis_index("s")
    wid = c * NUM_SUBCORES + s
    r0 = s * ROWS_PER_SUBCORE

    pltpu.sync_copy(zeros_hbm.at[pl.ds(r0, ROWS_PER_SUBCORE)],
                    agg_sh.at[pl.ds(r0, ROWS_PER_SUBCORE)])

    @pl.when(s == 0)
    def _():
        pltpu.sync_copy(zeros_hbm.at[pl.ds(TAIL_ROW0, TAIL_ROWS)],
                        agg_sh.at[pl.ds(TAIL_ROW0, TAIL_ROWS)])

    plsc.subcore_barrier()

    e_base = wid * EDGES_PER_TILE

    def body(j, carry):
        e0 = e_base + j * CHUNK
        pltpu.sync_copy(src_hbm.at[pl.ds(e0, CHUNK)], src_v)
        pltpu.sync_copy(dst_hbm.at[pl.ds(e0, CHUNK)], dst_v)
        pltpu.async_copy(h_hbm.at[src_v], rows_v, sem).wait()
        pltpu.sync_copy(rows_v, agg_sh.at[dst_v], add=True)
        return carry

    lax.fori_loop(0, N_CHUNKS, body, 0)
    plsc.subcore_barrier()

    pltpu.sync_copy(agg_sh.at[pl.ds(r0, ROWS_PER_SUBCORE)],
                    out_hbm.at[c, pl.ds(r0, ROWS_PER_SUBCORE)])

    @pl.when(s == 0)
    def _():
        pltpu.sync_copy(agg_sh.at[pl.ds(TAIL_ROW0, TAIL_ROWS)],
                        out_hbm.at[c, pl.ds(TAIL_ROW0, TAIL_ROWS)])


@functools.cache
def _get_sc_agg():
    return pl.kernel(
        _sc_agg_body,
        out_type=jax.ShapeDtypeStruct((NUM_CORES, N_NODES, DIM), jnp.float32),
        mesh=plsc.VectorSubcoreMesh(core_axis_name="c", subcore_axis_name="s",
                                    num_cores=NUM_CORES,
                                    num_subcores=NUM_SUBCORES),
        scratch_types=[
            pltpu.VMEM((CHUNK,), jnp.int32),
            pltpu.VMEM((CHUNK,), jnp.int32),
            pltpu.VMEM((CHUNK, DIM), jnp.float32),
            pltpu.VMEM_SHARED((N_NODES, DIM), jnp.float32),
            pltpu.SemaphoreType.DMA,
        ],
    )


def _bn(z, g, b):
    m = jnp.mean(z, axis=0, keepdims=True)
    v = jnp.mean((z - m) * (z - m), axis=0, keepdims=True)
    return (z - m) * lax.rsqrt(v + 1e-5) * g + b


def _tc_layer_body(h_ref, agg_ref, w1_ref, b1_ref, g1_ref, be1_ref,
                   w2_ref, b2_ref, g2_ref, be2_ref, out_ref):
    a = agg_ref[...]
    z = h_ref[...] + a[0] + a[1]
    z = jnp.dot(z, w1_ref[...], preferred_element_type=jnp.float32) + b1_ref[...]
    z = jnp.maximum(_bn(z, g1_ref[...], be1_ref[...]), 0.0)
    z = jnp.dot(z, w2_ref[...], preferred_element_type=jnp.float32) + b2_ref[...]
    z = jnp.maximum(_bn(z, g2_ref[...], be2_ref[...]), 0.0)
    out_ref[...] = z


def _tc_final_body(h_ref, agg_ref, w1_ref, b1_ref, g1_ref, be1_ref,
                   w2_ref, b2_ref, g2_ref, be2_ref,
                   batch_ref, clsw_ref, clsb_ref, out_ref):
    a = agg_ref[...]
    z = h_ref[...] + a[0] + a[1]
    z = jnp.dot(z, w1_ref[...], preferred_element_type=jnp.float32) + b1_ref[...]
    z = jnp.maximum(_bn(z, g1_ref[...], be1_ref[...]), 0.0)
    z = jnp.dot(z, w2_ref[...], preferred_element_type=jnp.float32) + b2_ref[...]
    z = _bn(z, g2_ref[...], be2_ref[...])

    ids = lax.broadcasted_iota(jnp.int32, (N_NODES, N_GRAPHS), 1)
    onehot = (batch_ref[...] == ids).astype(jnp.float32)
    dnums = (((0,), (0,)), ((), ()))
    sums = lax.dot_general(onehot, z, dnums,
                           preferred_element_type=jnp.float32)
    cnts = lax.dot_general(onehot, jnp.ones((N_NODES, 1), jnp.float32),
                           dnums, preferred_element_type=jnp.float32)
    readout = sums / jnp.maximum(cnts, 1.0)
    out_ref[...] = (jnp.dot(readout, clsw_ref[...],
                            preferred_element_type=jnp.float32)
                    + clsb_ref[...])


_tc_layer = pl.pallas_call(
    _tc_layer_body,
    out_shape=jax.ShapeDtypeStruct((N_NODES, DIM), jnp.float32),
)

_tc_final = pl.pallas_call(
    _tc_final_body,
    out_shape=jax.ShapeDtypeStruct((N_GRAPHS, N_OUT), jnp.float32),
)


def kernel(x, edge_index, batch, params):
    src = edge_index[0]
    dst = edge_index[1]
    zeros = jnp.zeros((N_NODES, DIM), jnp.float32)
    batch2d = batch.reshape(N_NODES, 1).astype(jnp.int32)

    h = x
    layers = params["layers"]
    out = None
    for i, p in enumerate(layers):
        aggs = _get_sc_agg()(h, src, dst, zeros)
        w = (p["W1"], p["b1"].reshape(1, -1), p["g1"].reshape(1, -1),
             p["be1"].reshape(1, -1), p["W2"], p["b2"].reshape(1, -1),
             p["g2"].reshape(1, -1), p["be2"].reshape(1, -1))
        if i != len(layers) - 1:
            h = _tc_layer(h, aggs, *w)
        else:
            out = _tc_final(h, aggs, *w, batch2d, params["cls_W"],
                            params["cls_b"].reshape(1, -1))
    return out


# speedup vs baseline: 7.9949x; 1.7562x over previous
import functools

import jax
import jax.numpy as jnp
from jax import lax
from jax.experimental import pallas as pl
from jax.experimental.pallas import tpu as pltpu
from jax.experimental.pallas import tpu_sc as plsc

N_NODES = 10000
N_EDGES = 320000
DIM = 128
N_GRAPHS = 64
N_OUT = 16

NUM_CORES = 2
NUM_SUBCORES = 16
NUM_TILES = NUM_CORES * NUM_SUBCORES
EDGES_PER_TILE = N_EDGES // NUM_TILES
CHUNK = 80
N_CHUNKS = EDGES_PER_TILE // CHUNK
N_PHASES = 5
CHUNKS_PER_PHASE = N_CHUNKS // N_PHASES
ROWS_PER_SUBCORE = 624
TAIL_ROW0 = ROWS_PER_SUBCORE * NUM_SUBCORES
TAIL_ROWS = N_NODES - TAIL_ROW0


def _sc_agg_body(h_hbm, src_hbm, dst_hbm, zeros_hbm, out_hbm,
                 src_p, dst_p, rows0, rows1, agg_sh,
                 sem_i, sem_g0, sem_g1):
    c = lax.axis_index("c")
    s = lax.axis_index("s")
    wid = c * NUM_SUBCORES + s
    r0 = s * ROWS_PER_SUBCORE

    pltpu.async_copy(src_hbm.at[wid, 0], src_p, sem_i)
    pltpu.async_copy(dst_hbm.at[wid, 0], dst_p, sem_i)

    pltpu.sync_copy(zeros_hbm.at[pl.ds(r0, ROWS_PER_SUBCORE)],
                    agg_sh.at[pl.ds(r0, ROWS_PER_SUBCORE)])

    @pl.when(s == 0)
    def _():
        pltpu.sync_copy(zeros_hbm.at[pl.ds(TAIL_ROW0, TAIL_ROWS)],
                        agg_sh.at[pl.ds(TAIL_ROW0, TAIL_ROWS)])

    pltpu.make_async_copy(src_hbm.at[wid, 0], src_p, sem_i).wait()
    pltpu.make_async_copy(dst_hbm.at[wid, 0], dst_p, sem_i).wait()
    plsc.subcore_barrier()

    def gather(j, rows, sem):
        pltpu.async_copy(h_hbm.at[src_p.at[j]], rows, sem)

    def gather_wait(rows, sem):
        pltpu.make_async_copy(h_hbm.at[src_p.at[0]], rows, sem).wait()

    def scatter(j, rows):
        pltpu.sync_copy(rows, agg_sh.at[dst_p.at[j]], add=True)

    def phase_body(ph, carry):
        gather(0, rows0, sem_g0)

        def pair_body(kk, carry):
            j0 = 2 * kk
            gather_wait(rows0, sem_g0)
            gather(j0 + 1, rows1, sem_g1)
            scatter(j0, rows0)
            gather_wait(rows1, sem_g1)

            @pl.when(j0 + 2 < CHUNKS_PER_PHASE)
            def _():
                gather(j0 + 2, rows0, sem_g0)

            scatter(j0 + 1, rows1)
            return carry

        lax.fori_loop(0, CHUNKS_PER_PHASE // 2, pair_body, 0)
        gather_wait(rows0, sem_g0)
        scatter(CHUNKS_PER_PHASE - 1, rows0)

        @pl.when(ph + 1 < N_PHASES)
        def _():
            pltpu.sync_copy(src_hbm.at[wid, ph + 1], src_p)
            pltpu.sync_copy(dst_hbm.at[wid, ph + 1], dst_p)

        return carry

    lax.fori_loop(0, N_PHASES, phase_body, 0)
    plsc.subcore_barrier()

    pltpu.sync_copy(agg_sh.at[pl.ds(r0, ROWS_PER_SUBCORE)],
                    out_hbm.at[c, pl.ds(r0, ROWS_PER_SUBCORE)])

    @pl.when(s == 0)
    def _():
        pltpu.sync_copy(agg_sh.at[pl.ds(TAIL_ROW0, TAIL_ROWS)],
                        out_hbm.at[c, pl.ds(TAIL_ROW0, TAIL_ROWS)])


@functools.cache
def _get_sc_agg():
    return pl.kernel(
        _sc_agg_body,
        out_type=jax.ShapeDtypeStruct((NUM_CORES, N_NODES, DIM), jnp.float32),
        mesh=plsc.VectorSubcoreMesh(core_axis_name="c", subcore_axis_name="s",
                                    num_cores=NUM_CORES,
                                    num_subcores=NUM_SUBCORES),
        scratch_types=[
            pltpu.VMEM((CHUNKS_PER_PHASE, CHUNK), jnp.int32),
            pltpu.VMEM((CHUNKS_PER_PHASE, CHUNK), jnp.int32),
            pltpu.VMEM((CHUNK, DIM), jnp.float32),
            pltpu.VMEM((CHUNK, DIM), jnp.float32),
            pltpu.VMEM_SHARED((N_NODES, DIM), jnp.float32),
            pltpu.SemaphoreType.DMA,
            pltpu.SemaphoreType.DMA,
            pltpu.SemaphoreType.DMA,
        ],
    )


def _bn(z, g, b):
    m = jnp.mean(z, axis=0, keepdims=True)
    v = jnp.mean((z - m) * (z - m), axis=0, keepdims=True)
    return (z - m) * lax.rsqrt(v + 1e-5) * g + b


def _tc_layer_body(h_ref, agg_ref, w1_ref, b1_ref, g1_ref, be1_ref,
                   w2_ref, b2_ref, g2_ref, be2_ref, out_ref):
    a = agg_ref[...]
    z = h_ref[...] + a[0] + a[1]
    z = jnp.dot(z, w1_ref[...], preferred_element_type=jnp.float32) + b1_ref[...]
    z = jnp.maximum(_bn(z, g1_ref[...], be1_ref[...]), 0.0)
    z = jnp.dot(z, w2_ref[...], preferred_element_type=jnp.float32) + b2_ref[...]
    z = jnp.maximum(_bn(z, g2_ref[...], be2_ref[...]), 0.0)
    out_ref[...] = z


def _tc_final_body(h_ref, agg_ref, w1_ref, b1_ref, g1_ref, be1_ref,
                   w2_ref, b2_ref, g2_ref, be2_ref,
                   batch_ref, clsw_ref, clsb_ref, out_ref):
    a = agg_ref[...]
    z = h_ref[...] + a[0] + a[1]
    z = jnp.dot(z, w1_ref[...], preferred_element_type=jnp.float32) + b1_ref[...]
    z = jnp.maximum(_bn(z, g1_ref[...], be1_ref[...]), 0.0)
    z = jnp.dot(z, w2_ref[...], preferred_element_type=jnp.float32) + b2_ref[...]
    z = _bn(z, g2_ref[...], be2_ref[...])

    ids = lax.broadcasted_iota(jnp.int32, (N_NODES, N_GRAPHS), 1)
    onehot = (batch_ref[...] == ids).astype(jnp.float32)
    dnums = (((0,), (0,)), ((), ()))
    sums = lax.dot_general(onehot, z, dnums,
                           preferred_element_type=jnp.float32)
    cnts = lax.dot_general(onehot, jnp.ones((N_NODES, 1), jnp.float32),
                           dnums, preferred_element_type=jnp.float32)
    readout = sums / jnp.maximum(cnts, 1.0)
    out_ref[...] = (jnp.dot(readout, clsw_ref[...],
                            preferred_element_type=jnp.float32)
                    + clsb_ref[...])


_tc_layer = pl.pallas_call(
    _tc_layer_body,
    out_shape=jax.ShapeDtypeStruct((N_NODES, DIM), jnp.float32),
)

_tc_final = pl.pallas_call(
    _tc_final_body,
    out_shape=jax.ShapeDtypeStruct((N_GRAPHS, N_OUT), jnp.float32),
)


def kernel(x, edge_index, batch, params):
    src = edge_index[0]
    dst = edge_index[1]
    src3 = src.reshape(NUM_TILES, N_PHASES, CHUNKS_PER_PHASE, CHUNK)
    dst3 = dst.reshape(NUM_TILES, N_PHASES, CHUNKS_PER_PHASE, CHUNK)
    zeros = jnp.zeros((N_NODES, DIM), jnp.float32)
    batch2d = batch.reshape(N_NODES, 1).astype(jnp.int32)

    h = x
    layers = params["layers"]
    out = None
    for i, p in enumerate(layers):
        aggs = _get_sc_agg()(h, src3, dst3, zeros)
        w = (p["W1"], p["b1"].reshape(1, -1), p["g1"].reshape(1, -1),
             p["be1"].reshape(1, -1), p["W2"], p["b2"].reshape(1, -1),
             p["g2"].reshape(1, -1), p["be2"].reshape(1, -1))
        if i != len(layers) - 1:
            h = _tc_layer(h, aggs, *w)
        else:
            out = _tc_final(h, aggs, *w, batch2d, params["cls_W"],
                            params["cls_b"].reshape(1, -1))
    return out


# speedup vs baseline: 11.2046x; 1.4015x over previous
import functools

import jax
import jax.numpy as jnp
from jax import lax
from jax.experimental import pallas as pl
from jax.experimental.pallas import tpu as pltpu
from jax.experimental.pallas import tpu_sc as plsc

N_NODES = 10000
N_EDGES = 320000
DIM = 128
N_GRAPHS = 64
N_OUT = 16

NUM_CORES = 2
NUM_SUBCORES = 16
NUM_TILES = NUM_CORES * NUM_SUBCORES
EDGES_PER_TILE = N_EDGES // NUM_TILES
CHUNK = 80
N_CHUNKS = EDGES_PER_TILE // CHUNK
N_PHASES = 5
CHUNKS_PER_PHASE = N_CHUNKS // N_PHASES
ROWS_PER_SUBCORE = 624
TAIL_ROW0 = ROWS_PER_SUBCORE * NUM_SUBCORES
TAIL_ROWS = N_NODES - TAIL_ROW0


def _sc_agg_body(h_hbm, src_hbm, dst_hbm, zeros_hbm, out_hbm,
                 src_p, dst_p, rows0, rows1, rows2, rows3, agg_sh,
                 sem_i, sem_g0, sem_g1, sem_g2, sem_g3):
    c = lax.axis_index("c")
    s = lax.axis_index("s")
    wid = c * NUM_SUBCORES + s
    r0 = s * ROWS_PER_SUBCORE

    pltpu.async_copy(src_hbm.at[wid, 0], src_p, sem_i)
    pltpu.async_copy(dst_hbm.at[wid, 0], dst_p, sem_i)

    pltpu.sync_copy(zeros_hbm.at[pl.ds(r0, ROWS_PER_SUBCORE)],
                    agg_sh.at[pl.ds(r0, ROWS_PER_SUBCORE)])

    @pl.when(s == 0)
    def _():
        pltpu.sync_copy(zeros_hbm.at[pl.ds(TAIL_ROW0, TAIL_ROWS)],
                        agg_sh.at[pl.ds(TAIL_ROW0, TAIL_ROWS)])

    pltpu.make_async_copy(src_hbm.at[wid, 0], src_p, sem_i).wait()
    pltpu.make_async_copy(dst_hbm.at[wid, 0], dst_p, sem_i).wait()
    plsc.subcore_barrier()

    def gather(j, rows, sem):
        pltpu.async_copy(h_hbm.at[src_p.at[j]], rows, sem)

    def gather_wait(rows, sem):
        pltpu.make_async_copy(h_hbm.at[src_p.at[0]], rows, sem).wait()

    def scatter(j, rows):
        pltpu.sync_copy(rows, agg_sh.at[dst_p.at[j]], add=True)

    bufs = ((rows0, sem_g0), (rows1, sem_g1), (rows2, sem_g2),
            (rows3, sem_g3))

    def phase_body(ph, carry):
        for b, (rows, sem) in enumerate(bufs):
            gather(b, rows, sem)

        def group_body(kk, carry):
            j0 = 4 * kk
            for b, (rows, sem) in enumerate(bufs):
                gather_wait(rows, sem)
                scatter(j0 + b, rows)

                @pl.when(j0 + b + 4 < CHUNKS_PER_PHASE)
                def _(rows=rows, sem=sem, b=b):
                    gather(j0 + b + 4, rows, sem)

            return carry

        lax.fori_loop(0, CHUNKS_PER_PHASE // 4, group_body, 0)
        gather_wait(rows0, sem_g0)
        scatter(CHUNKS_PER_PHASE - 1, rows0)

        @pl.when(ph + 1 < N_PHASES)
        def _():
            pltpu.sync_copy(src_hbm.at[wid, ph + 1], src_p)
            pltpu.sync_copy(dst_hbm.at[wid, ph + 1], dst_p)

        return carry

    lax.fori_loop(0, N_PHASES, phase_body, 0)
    plsc.subcore_barrier()

    pltpu.sync_copy(agg_sh.at[pl.ds(r0, ROWS_PER_SUBCORE)],
                    out_hbm.at[c, pl.ds(r0, ROWS_PER_SUBCORE)])

    @pl.when(s == 0)
    def _():
        pltpu.sync_copy(agg_sh.at[pl.ds(TAIL_ROW0, TAIL_ROWS)],
                        out_hbm.at[c, pl.ds(TAIL_ROW0, TAIL_ROWS)])


@functools.cache
def _get_sc_agg():
    return pl.kernel(
        _sc_agg_body,
        out_type=jax.ShapeDtypeStruct((NUM_CORES, N_NODES, DIM), jnp.float32),
        mesh=plsc.VectorSubcoreMesh(core_axis_name="c", subcore_axis_name="s",
                                    num_cores=NUM_CORES,
                                    num_subcores=NUM_SUBCORES),
        scratch_types=[
            pltpu.VMEM((CHUNKS_PER_PHASE, CHUNK), jnp.int32),
            pltpu.VMEM((CHUNKS_PER_PHASE, CHUNK), jnp.int32),
            pltpu.VMEM((CHUNK, DIM), jnp.float32),
            pltpu.VMEM((CHUNK, DIM), jnp.float32),
            pltpu.VMEM((CHUNK, DIM), jnp.float32),
            pltpu.VMEM((CHUNK, DIM), jnp.float32),
            pltpu.VMEM_SHARED((N_NODES, DIM), jnp.float32),
            pltpu.SemaphoreType.DMA,
            pltpu.SemaphoreType.DMA,
            pltpu.SemaphoreType.DMA,
            pltpu.SemaphoreType.DMA,
            pltpu.SemaphoreType.DMA,
        ],
    )


def _bn(z, g, b):
    m = jnp.mean(z, axis=0, keepdims=True)
    v = jnp.mean((z - m) * (z - m), axis=0, keepdims=True)
    return (z - m) * lax.rsqrt(v + 1e-5) * g + b


def _tc_layer_body(h_ref, agg_ref, w1_ref, b1_ref, g1_ref, be1_ref,
                   w2_ref, b2_ref, g2_ref, be2_ref, out_ref):
    a = agg_ref[...]
    z = h_ref[...] + a[0] + a[1]
    z = jnp.dot(z, w1_ref[...], preferred_element_type=jnp.float32) + b1_ref[...]
    z = jnp.maximum(_bn(z, g1_ref[...], be1_ref[...]), 0.0)
    z = jnp.dot(z, w2_ref[...], preferred_element_type=jnp.float32) + b2_ref[...]
    z = jnp.maximum(_bn(z, g2_ref[...], be2_ref[...]), 0.0)
    out_ref[...] = z


def _tc_final_body(h_ref, agg_ref, w1_ref, b1_ref, g1_ref, be1_ref,
                   w2_ref, b2_ref, g2_ref, be2_ref,
                   batch_ref, clsw_ref, clsb_ref, out_ref):
    a = agg_ref[...]
    z = h_ref[...] + a[0] + a[1]
    z = jnp.dot(z, w1_ref[...], preferred_element_type=jnp.float32) + b1_ref[...]
    z = jnp.maximum(_bn(z, g1_ref[...], be1_ref[...]), 0.0)
    z = jnp.dot(z, w2_ref[...], preferred_element_type=jnp.float32) + b2_ref[...]
    z = _bn(z, g2_ref[...], be2_ref[...])

    ids = lax.broadcasted_iota(jnp.int32, (N_NODES, N_GRAPHS), 1)
    onehot = (batch_ref[...] == ids).astype(jnp.float32)
    dnums = (((0,), (0,)), ((), ()))
    sums = lax.dot_general(onehot, z, dnums,
                           preferred_element_type=jnp.float32)
    cnts = lax.dot_general(onehot, jnp.ones((N_NODES, 1), jnp.float32),
                           dnums, preferred_element_type=jnp.float32)
    readout = sums / jnp.maximum(cnts, 1.0)
    out_ref[...] = (jnp.dot(readout, clsw_ref[...],
                            preferred_element_type=jnp.float32)
                    + clsb_ref[...])


_tc_layer = pl.pallas_call(
    _tc_layer_body,
    out_shape=jax.ShapeDtypeStruct((N_NODES, DIM), jnp.float32),
)

_tc_final = pl.pallas_call(
    _tc_final_body,
    out_shape=jax.ShapeDtypeStruct((N_GRAPHS, N_OUT), jnp.float32),
)


def kernel(x, edge_index, batch, params):
    src = edge_index[0]
    dst = edge_index[1]
    src3 = src.reshape(NUM_TILES, N_PHASES, CHUNKS_PER_PHASE, CHUNK)
    dst3 = dst.reshape(NUM_TILES, N_PHASES, CHUNKS_PER_PHASE, CHUNK)
    zeros = jnp.zeros((N_NODES, DIM), jnp.float32)
    batch2d = batch.reshape(N_NODES, 1).astype(jnp.int32)

    h = x
    layers = params["layers"]
    out = None
    for i, p in enumerate(layers):
        aggs = _get_sc_agg()(h, src3, dst3, zeros)
        w = (p["W1"], p["b1"].reshape(1, -1), p["g1"].reshape(1, -1),
             p["be1"].reshape(1, -1), p["W2"], p["b2"].reshape(1, -1),
             p["g2"].reshape(1, -1), p["be2"].reshape(1, -1))
        if i != len(layers) - 1:
            h = _tc_layer(h, aggs, *w)
        else:
            out = _tc_final(h, aggs, *w, batch2d, params["cls_W"],
                            params["cls_b"].reshape(1, -1))
    return out
